# baseline (speedup 1.0000x reference)
import functools
import math

import jax
import jax.numpy as jnp
import numpy as np
from jax import lax
from jax.experimental import pallas as pl
from jax.experimental.pallas import tpu as pltpu
from jax.experimental.pallas import tpu_sc as plsc

N = 10000
E = 160000
IN_CH = 128
HID = 64
FILTER_NUM = 5
K = 5

NTILES = 32
EPT = 5120
EP = NTILES * EPT
EROWS = EP // 128
TROWS = EPT // 128
TRASH = N
NT = N + 112
STRIPE = NT // 16

ROWBLK = 400
GBLK = 400


def _bern_matrix(degree):
    cm = np.zeros((degree + 1, degree + 1), dtype=np.float64)
    for i in range(degree + 1):
        for j in range(degree - i + 1):
            cm[i, i + j] = math.comb(degree, i) * math.comb(degree - i, j) * ((-1.0) ** j)
    return cm


_CM = _bern_matrix(K).astype(np.float32)


def _zero_fill(buf, rows, width):
    zero16 = jnp.zeros((16,), dtype=jnp.float32)

    def body(i, _):
        for q in range(width // 16):
            buf[i, pl.ds(q * 16, 16)] = zero16
        return 0

    lax.fori_loop(0, rows, body, 0)


def _zero_stripe(zbuf, acc, sid):
    base = sid * STRIPE
    for off in (0, 128, 256, 384, STRIPE - 128):
        pltpu.sync_copy(zbuf, acc.at[pl.ds(base + off, 128)])


def _sc_prep_body(rowp_hbm, colp_hbm, scol_hbm, degp_hbm,
                  rvm, cvm, srvm, scvm, ones_v, dacc, dsem):
    cid = lax.axis_index("c")
    sid = lax.axis_index("s")
    wid = sid * 2 + cid
    base = wid * TROWS

    pltpu.sync_copy(rowp_hbm.at[pl.ds(base, TROWS)], rvm)
    pltpu.sync_copy(colp_hbm.at[pl.ds(base, TROWS)], cvm)

    _zero_fill(ones_v, 128, HID)
    _zero_stripe(ones_v, dacc, sid)

    one16 = jnp.full((16,), 1.0, dtype=jnp.float32)

    def fill_ones(i, _):
        for q in range(HID // 16):
            ones_v[i, pl.ds(q * 16, 16)] = one16
        return 0

    lax.fori_loop(0, 128, fill_ones, 0)

    trash = jnp.full((16,), TRASH, dtype=jnp.int32)

    def fixup(j, _):
        for q in range(8):
            sl = pl.ds(q * 16, 16)
            r = rvm[j, sl]
            c = cvm[j, sl]
            m = r == c
            t = trash + (r & 63)
            srvm[j, sl] = jnp.where(m, t, r)
            scvm[j, sl] = jnp.where(m, t, c)
        return 0

    lax.fori_loop(0, TROWS, fixup, 0)

    pltpu.sync_copy(scvm, scol_hbm.at[pl.ds(base, TROWS)])

    plsc.subcore_barrier()

    handles = [pltpu.async_copy(ones_v, dacc.at[srvm.at[j]], dsem, add=True)
               for j in range(TROWS)]
    for h in handles:
        h.wait()

    plsc.subcore_barrier()
    pltpu.sync_copy(dacc.at[pl.ds(sid * STRIPE, STRIPE)],
                    degp_hbm.at[cid, pl.ds(sid * STRIPE, STRIPE)])


def _sc_round_body(g_hbm, rowp_hbm, scol_hbm, out_hbm,
                   ridx, cidx, buf_a, buf_b, buf_c, buf_d, acc, sem, sem2):
    cid = lax.axis_index("c")
    sid = lax.axis_index("s")
    wid = sid * 2 + cid
    base = wid * TROWS

    pltpu.sync_copy(rowp_hbm.at[pl.ds(base, TROWS)], ridx)
    pltpu.sync_copy(scol_hbm.at[pl.ds(base, TROWS)], cidx)

    _zero_fill(buf_a, 128, HID)
    _zero_stripe(buf_a, acc, sid)
    plsc.subcore_barrier()

    bufs = (buf_a, buf_b, buf_c, buf_d)
    gh, sc = {}, {}
    gh[0] = pltpu.async_copy(g_hbm.at[ridx.at[0]], bufs[0], sem)
    gh[1] = pltpu.async_copy(g_hbm.at[ridx.at[1]], bufs[1], sem)
    for j in range(TROWS):
        gh[j].wait()
        if j + 2 < TROWS:
            if j - 2 >= 0:
                sc[j - 2].wait()
            gh[j + 2] = pltpu.async_copy(g_hbm.at[ridx.at[j + 2]],
                                         bufs[(j + 2) % 4], sem)
        sc[j] = pltpu.async_copy(bufs[j % 4], acc.at[cidx.at[j]], sem2,
                                 add=True)
    for j in range(TROWS - 4, TROWS):
        sc[j].wait()

    plsc.subcore_barrier()
    pltpu.sync_copy(acc.at[pl.ds(sid * STRIPE, STRIPE)],
                    out_hbm.at[cid, pl.ds(sid * STRIPE, STRIPE)])


@functools.lru_cache(maxsize=None)
def _sc_kernels():
    mesh = plsc.VectorSubcoreMesh(core_axis_name="c", subcore_axis_name="s")
    prep = pl.kernel(
        _sc_prep_body,
        out_type=(
            jax.ShapeDtypeStruct((EROWS, 128), jnp.int32),
            jax.ShapeDtypeStruct((2, NT, HID), jnp.float32),
        ),
        mesh=mesh,
        compiler_params=pltpu.CompilerParams(use_tc_tiling_on_sc=False),
        scratch_types=[
            pltpu.VMEM((TROWS, 128), jnp.int32),
            pltpu.VMEM((TROWS, 128), jnp.int32),
            pltpu.VMEM((TROWS, 128), jnp.int32),
            pltpu.VMEM((TROWS, 128), jnp.int32),
            pltpu.VMEM((128, HID), jnp.float32),
            pltpu.VMEM_SHARED((NT, HID), jnp.float32),
            pltpu.SemaphoreType.DMA,
        ],
    )
    rnd = pl.kernel(
        _sc_round_body,
        out_type=jax.ShapeDtypeStruct((2, NT, HID), jnp.float32),
        mesh=mesh,
        compiler_params=pltpu.CompilerParams(use_tc_tiling_on_sc=False),
        scratch_types=[
            pltpu.VMEM((TROWS, 128), jnp.int32),
            pltpu.VMEM((TROWS, 128), jnp.int32),
            pltpu.VMEM((128, HID), jnp.float32),
            pltpu.VMEM((128, HID), jnp.float32),
            pltpu.VMEM((128, HID), jnp.float32),
            pltpu.VMEM((128, HID), jnp.float32),
            pltpu.VMEM_SHARED((NT, HID), jnp.float32),
            pltpu.SemaphoreType.DMA,
            pltpu.SemaphoreType.DMA,
        ],
    )
    return prep, rnd


def _mlp_body(x_ref, w1_ref, b1_ref, w2_ref, b2_ref, degp_ref,
              h0_ref, g0_ref, dinv_ref):
    h = jnp.dot(x_ref[...], w1_ref[...], preferred_element_type=jnp.float32)
    h = jnp.maximum(h + b1_ref[...], 0.0)
    h = jnp.dot(h, w2_ref[...], preferred_element_type=jnp.float32) + b2_ref[...]
    dp = degp_ref[...]
    deg = (dp[0] + dp[1])[:, 0:1]
    dinv = jnp.where(deg > 0.0, lax.rsqrt(deg), 0.0)
    h0_ref[...] = h
    g0_ref[...] = dinv * h
    dinv_ref[...] = jnp.broadcast_to(dinv, (h.shape[0], HID))


_mlp_call = pl.pallas_call(
    _mlp_body,
    grid=(N // ROWBLK,),
    in_specs=[
        pl.BlockSpec((ROWBLK, IN_CH), lambda i: (i, 0)),
        pl.BlockSpec((IN_CH, HID), lambda i: (0, 0)),
        pl.BlockSpec((1, HID), lambda i: (0, 0)),
        pl.BlockSpec((HID, HID), lambda i: (0, 0)),
        pl.BlockSpec((1, HID), lambda i: (0, 0)),
        pl.BlockSpec((2, ROWBLK, HID), lambda i: (0, i, 0)),
    ],
    out_specs=[
        pl.BlockSpec((ROWBLK, HID), lambda i: (i, 0)),
        pl.BlockSpec((ROWBLK, HID), lambda i: (i, 0)),
        pl.BlockSpec((ROWBLK, HID), lambda i: (i, 0)),
    ],
    out_shape=[
        jax.ShapeDtypeStruct((N, HID), jnp.float32),
        jax.ShapeDtypeStruct((N, HID), jnp.float32),
        jax.ShapeDtypeStruct((N, HID), jnp.float32),
    ],
)


def _combine_body(bx_ref, p_ref, dinv_ref, bn_ref, gn_ref):
    p = p_ref[...]
    s = p[0] + p[1]
    dinv = dinv_ref[...]
    bn = 0.5 * bx_ref[...] - 0.5 * dinv * s
    bn_ref[...] = bn
    gn_ref[...] = dinv * bn


_combine_call = pl.pallas_call(
    _combine_body,
    grid=(N // ROWBLK,),
    in_specs=[
        pl.BlockSpec((ROWBLK, HID), lambda i: (i, 0)),
        pl.BlockSpec((2, ROWBLK, HID), lambda i: (0, i, 0)),
        pl.BlockSpec((ROWBLK, HID), lambda i: (i, 0)),
    ],
    out_specs=[
        pl.BlockSpec((ROWBLK, HID), lambda i: (i, 0)),
        pl.BlockSpec((ROWBLK, HID), lambda i: (i, 0)),
    ],
    out_shape=[
        jax.ShapeDtypeStruct((N, HID), jnp.float32),
        jax.ShapeDtypeStruct((N, HID), jnp.float32),
    ],
)


def _attn_body(b0_ref, b1_ref, b2_ref, b3_ref, b4_ref, b5_ref,
               fw_ref, cm_ref, wf_ref, bf_ref, wx_ref, bxb_ref, res_ref):
    bs = [b0_ref[...], b1_ref[...], b2_ref[...], b3_ref[...],
          b4_ref[...], b5_ref[...]]
    sig = jax.nn.sigmoid(fw_ref[...])
    c2 = jnp.dot(sig, cm_ref[...], preferred_element_type=jnp.float32)

    xp = jnp.tanh(jnp.dot(bs[0], wx_ref[...],
                          preferred_element_type=jnp.float32) + bxb_ref[...])

    hfs, logits = [], []
    for f in range(FILTER_NUM):
        hf = bs[0] * c2[f:f + 1, 0:1]
        for i in range(1, K + 1):
            hf = hf + bs[i] * c2[f:f + 1, i:i + 1]
        hp = jnp.tanh(jnp.dot(hf, wf_ref[...],
                              preferred_element_type=jnp.float32) + bf_ref[...])
        hfs.append(hf)
        logits.append(jnp.sum(hp * xp, axis=1, keepdims=True))

    m = logits[0]
    for f in range(1, FILTER_NUM):
        m = jnp.maximum(m, logits[f])
    exps = [jnp.exp(l - m) for l in logits]
    denom = exps[0]
    for f in range(1, FILTER_NUM):
        denom = denom + exps[f]
    res = hfs[0] * (exps[0] / denom)
    for f in range(1, FILTER_NUM):
        res = res + hfs[f] * (exps[f] / denom)
    res_ref[...] = res.astype(jnp.bfloat16)


_attn_call = pl.pallas_call(
    _attn_body,
    grid=(N // ROWBLK,),
    in_specs=[pl.BlockSpec((ROWBLK, HID), lambda i: (i, 0))] * 6 + [
        pl.BlockSpec((FILTER_NUM, K + 1), lambda i: (0, 0)),
        pl.BlockSpec((K + 1, K + 1), lambda i: (0, 0)),
        pl.BlockSpec((HID, HID), lambda i: (0, 0)),
        pl.BlockSpec((1, HID), lambda i: (0, 0)),
        pl.BlockSpec((HID, HID), lambda i: (0, 0)),
        pl.BlockSpec((1, HID), lambda i: (0, 0)),
    ],
    out_specs=pl.BlockSpec((ROWBLK, HID), lambda i: (i, 0)),
    out_shape=jax.ShapeDtypeStruct((N, HID), jnp.bfloat16),
)


def _gram_body(a_ref, b_ref, o_ref):
    o_ref[...] = lax.dot_general(
        a_ref[...], b_ref[...], (((1,), (1,)), ((), ())),
        preferred_element_type=jnp.float32)


_gram_call = pl.pallas_call(
    _gram_body,
    grid=(N // GBLK,),
    in_specs=[
        pl.BlockSpec((GBLK, HID), lambda i: (i, 0)),
        pl.BlockSpec((N, HID), lambda i: (0, 0)),
    ],
    out_specs=pl.BlockSpec((GBLK, N), lambda i: (i, 0)),
    out_shape=jax.ShapeDtypeStruct((N, N), jnp.float32),
)


def kernel(x, edge_index, W1, b1, W2, b2, filt_w, Wf, bf, Wx, bx):
    pad = jnp.arange(EP - E, dtype=jnp.int32) % N
    rowp = jnp.concatenate([edge_index[0], pad]).reshape(EROWS, 128)
    colp = jnp.concatenate([edge_index[1], pad]).reshape(EROWS, 128)

    _sc_prep, _sc_round = _sc_kernels()
    scol, degp = _sc_prep(rowp, colp)
    h0, g, dinvb = _mlp_call(x, W1, b1.reshape(1, HID), W2,
                             b2.reshape(1, HID), degp)

    bs = [h0]
    for _ in range(K):
        p = _sc_round(g, rowp, scol)
        bn, g = _combine_call(bs[-1], p, dinvb)
        bs.append(bn)

    res = _attn_call(*bs, filt_w, jnp.asarray(_CM), Wf, bf.reshape(1, HID),
                     Wx, bx.reshape(1, HID))
    return _gram_call(res, res)


# speedup vs baseline: 1.0020x; 1.0020x over previous
import functools
import math

import jax
import jax.numpy as jnp
import numpy as np
from jax import lax
from jax.experimental import pallas as pl
from jax.experimental.pallas import tpu as pltpu
from jax.experimental.pallas import tpu_sc as plsc

N = 10000
E = 160000
IN_CH = 128
HID = 64
FILTER_NUM = 5
K = 5

NTILES = 32
EPT = 5120
EP = NTILES * EPT
EROWS = EP // 128
TROWS = EPT // 128
TRASH = N
NT = N + 112
STRIPE = NT // 16

ROWBLK = 400
GBLK = 400


def _bern_matrix(degree):
    cm = np.zeros((degree + 1, degree + 1), dtype=np.float64)
    for i in range(degree + 1):
        for j in range(degree - i + 1):
            cm[i, i + j] = math.comb(degree, i) * math.comb(degree - i, j) * ((-1.0) ** j)
    return cm


_CM = _bern_matrix(K).astype(np.float32)


def _zero_fill(buf, rows, width):
    zero16 = jnp.zeros((16,), dtype=jnp.float32)

    def body(i, _):
        for q in range(width // 16):
            buf[i, pl.ds(q * 16, 16)] = zero16
        return 0

    lax.fori_loop(0, rows, body, 0)


def _zero_stripe(zbuf, acc, sid):
    base = sid * STRIPE
    for off in (0, 128, 256, 384, STRIPE - 128):
        pltpu.sync_copy(zbuf, acc.at[pl.ds(base + off, 128)])


def _sc_prep_body(rowp_hbm, colp_hbm, scol_hbm, degp_hbm,
                  rvm, cvm, srvm, scvm, ones_v, dacc, dsem):
    cid = lax.axis_index("c")
    sid = lax.axis_index("s")
    wid = sid * 2 + cid
    base = wid * TROWS

    pltpu.sync_copy(rowp_hbm.at[pl.ds(base, TROWS)], rvm)
    pltpu.sync_copy(colp_hbm.at[pl.ds(base, TROWS)], cvm)

    _zero_fill(ones_v, 128, HID)
    _zero_stripe(ones_v, dacc, sid)

    one16 = jnp.full((16,), 1.0, dtype=jnp.float32)

    def fill_ones(i, _):
        for q in range(HID // 16):
            ones_v[i, pl.ds(q * 16, 16)] = one16
        return 0

    lax.fori_loop(0, 128, fill_ones, 0)

    trash = jnp.full((16,), TRASH, dtype=jnp.int32)

    def fixup(j, _):
        for q in range(8):
            sl = pl.ds(q * 16, 16)
            r = rvm[j, sl]
            c = cvm[j, sl]
            m = r == c
            t = trash + (r & 63)
            srvm[j, sl] = jnp.where(m, t, r)
            scvm[j, sl] = jnp.where(m, t, c)
        return 0

    lax.fori_loop(0, TROWS, fixup, 0)

    pltpu.sync_copy(scvm, scol_hbm.at[pl.ds(base, TROWS)])

    plsc.subcore_barrier()

    handles = [pltpu.async_copy(ones_v, dacc.at[srvm.at[j]], dsem, add=True)
               for j in range(TROWS)]
    for h in handles:
        h.wait()

    plsc.subcore_barrier()
    pltpu.sync_copy(dacc.at[pl.ds(sid * STRIPE, STRIPE)],
                    degp_hbm.at[cid, pl.ds(sid * STRIPE, STRIPE)])


def _sc_round_body(g_hbm, rowp_hbm, scol_hbm, out_hbm,
                   ridx, cidx, buf_a, buf_b, buf_c, buf_d, acc, sem, sem2):
    cid = lax.axis_index("c")
    sid = lax.axis_index("s")
    wid = sid * 2 + cid
    base = wid * TROWS

    pltpu.sync_copy(rowp_hbm.at[pl.ds(base, TROWS)], ridx)
    pltpu.sync_copy(scol_hbm.at[pl.ds(base, TROWS)], cidx)

    _zero_fill(buf_a, 128, HID)
    _zero_stripe(buf_a, acc, sid)
    plsc.subcore_barrier()

    bufs = (buf_a, buf_b, buf_c, buf_d)
    gh, sc = {}, {}
    gh[0] = pltpu.async_copy(g_hbm.at[ridx.at[0]], bufs[0], sem)
    gh[1] = pltpu.async_copy(g_hbm.at[ridx.at[1]], bufs[1], sem)
    for j in range(TROWS):
        gh[j].wait()
        if j + 2 < TROWS:
            if j - 2 >= 0:
                sc[j - 2].wait()
            gh[j + 2] = pltpu.async_copy(g_hbm.at[ridx.at[j + 2]],
                                         bufs[(j + 2) % 4], sem)
        sc[j] = pltpu.async_copy(bufs[j % 4], acc.at[cidx.at[j]], sem2,
                                 add=True)
    for j in range(TROWS - 4, TROWS):
        sc[j].wait()

    plsc.subcore_barrier()
    pltpu.sync_copy(acc.at[pl.ds(sid * STRIPE, STRIPE)],
                    out_hbm.at[cid, pl.ds(sid * STRIPE, STRIPE)])


@functools.lru_cache(maxsize=None)
def _sc_kernels():
    mesh = plsc.VectorSubcoreMesh(core_axis_name="c", subcore_axis_name="s")
    prep = pl.kernel(
        _sc_prep_body,
        out_type=(
            jax.ShapeDtypeStruct((EROWS, 128), jnp.int32),
            jax.ShapeDtypeStruct((2, NT, HID), jnp.float32),
        ),
        mesh=mesh,
        compiler_params=pltpu.CompilerParams(use_tc_tiling_on_sc=False),
        scratch_types=[
            pltpu.VMEM((TROWS, 128), jnp.int32),
            pltpu.VMEM((TROWS, 128), jnp.int32),
            pltpu.VMEM((TROWS, 128), jnp.int32),
            pltpu.VMEM((TROWS, 128), jnp.int32),
            pltpu.VMEM((128, HID), jnp.float32),
            pltpu.VMEM_SHARED((NT, HID), jnp.float32),
            pltpu.SemaphoreType.DMA,
        ],
    )
    rnd = pl.kernel(
        _sc_round_body,
        out_type=jax.ShapeDtypeStruct((2, NT, HID), jnp.float32),
        mesh=mesh,
        compiler_params=pltpu.CompilerParams(use_tc_tiling_on_sc=False),
        scratch_types=[
            pltpu.VMEM((TROWS, 128), jnp.int32),
            pltpu.VMEM((TROWS, 128), jnp.int32),
            pltpu.VMEM((128, HID), jnp.float32),
            pltpu.VMEM((128, HID), jnp.float32),
            pltpu.VMEM((128, HID), jnp.float32),
            pltpu.VMEM((128, HID), jnp.float32),
            pltpu.VMEM_SHARED((NT, HID), jnp.float32),
            pltpu.SemaphoreType.DMA,
            pltpu.SemaphoreType.DMA,
        ],
    )
    return prep, rnd


def _mlp_body(x_ref, w1_ref, b1_ref, w2_ref, b2_ref, degp_ref,
              h0_ref, g0_ref, dinv_ref):
    h = jnp.dot(x_ref[...], w1_ref[...], preferred_element_type=jnp.float32)
    h = jnp.maximum(h + b1_ref[...], 0.0)
    h = jnp.dot(h, w2_ref[...], preferred_element_type=jnp.float32) + b2_ref[...]
    dp = degp_ref[...]
    deg = (dp[0] + dp[1])[:, 0:1]
    dinv = jnp.where(deg > 0.0, lax.rsqrt(deg), 0.0)
    h0_ref[...] = h
    g0_ref[...] = dinv * h
    dinv_ref[...] = jnp.broadcast_to(dinv, (h.shape[0], HID))


_mlp_call = pl.pallas_call(
    _mlp_body,
    grid=(N // ROWBLK,),
    in_specs=[
        pl.BlockSpec((ROWBLK, IN_CH), lambda i: (i, 0)),
        pl.BlockSpec((IN_CH, HID), lambda i: (0, 0)),
        pl.BlockSpec((1, HID), lambda i: (0, 0)),
        pl.BlockSpec((HID, HID), lambda i: (0, 0)),
        pl.BlockSpec((1, HID), lambda i: (0, 0)),
        pl.BlockSpec((2, ROWBLK, HID), lambda i: (0, i, 0)),
    ],
    out_specs=[
        pl.BlockSpec((ROWBLK, HID), lambda i: (i, 0)),
        pl.BlockSpec((ROWBLK, HID), lambda i: (i, 0)),
        pl.BlockSpec((ROWBLK, HID), lambda i: (i, 0)),
    ],
    out_shape=[
        jax.ShapeDtypeStruct((N, HID), jnp.float32),
        jax.ShapeDtypeStruct((N, HID), jnp.float32),
        jax.ShapeDtypeStruct((N, HID), jnp.float32),
    ],
)


def _combine_body(bx_ref, p_ref, dinv_ref, bn_ref, gn_ref):
    p = p_ref[...]
    s = p[0] + p[1]
    dinv = dinv_ref[...]
    bn = 0.5 * bx_ref[...] - 0.5 * dinv * s
    bn_ref[...] = bn
    gn_ref[...] = dinv * bn


_combine_call = pl.pallas_call(
    _combine_body,
    grid=(N // ROWBLK,),
    in_specs=[
        pl.BlockSpec((ROWBLK, HID), lambda i: (i, 0)),
        pl.BlockSpec((2, ROWBLK, HID), lambda i: (0, i, 0)),
        pl.BlockSpec((ROWBLK, HID), lambda i: (i, 0)),
    ],
    out_specs=[
        pl.BlockSpec((ROWBLK, HID), lambda i: (i, 0)),
        pl.BlockSpec((ROWBLK, HID), lambda i: (i, 0)),
    ],
    out_shape=[
        jax.ShapeDtypeStruct((N, HID), jnp.float32),
        jax.ShapeDtypeStruct((N, HID), jnp.float32),
    ],
)


def _attn_body(b0_ref, b1_ref, b2_ref, b3_ref, b4_ref, b5_ref,
               fw_ref, cm_ref, wf_ref, bf_ref, wx_ref, bxb_ref, res_ref):
    bs = [b0_ref[...], b1_ref[...], b2_ref[...], b3_ref[...],
          b4_ref[...], b5_ref[...]]
    sig = jax.nn.sigmoid(fw_ref[...])
    c2 = jnp.dot(sig, cm_ref[...], preferred_element_type=jnp.float32)

    xp = jnp.tanh(jnp.dot(bs[0], wx_ref[...],
                          preferred_element_type=jnp.float32) + bxb_ref[...])

    hfs, logits = [], []
    for f in range(FILTER_NUM):
        hf = bs[0] * c2[f:f + 1, 0:1]
        for i in range(1, K + 1):
            hf = hf + bs[i] * c2[f:f + 1, i:i + 1]
        hp = jnp.tanh(jnp.dot(hf, wf_ref[...],
                              preferred_element_type=jnp.float32) + bf_ref[...])
        hfs.append(hf)
        logits.append(jnp.sum(hp * xp, axis=1, keepdims=True))

    m = logits[0]
    for f in range(1, FILTER_NUM):
        m = jnp.maximum(m, logits[f])
    exps = [jnp.exp(l - m) for l in logits]
    denom = exps[0]
    for f in range(1, FILTER_NUM):
        denom = denom + exps[f]
    res = hfs[0] * (exps[0] / denom)
    for f in range(1, FILTER_NUM):
        res = res + hfs[f] * (exps[f] / denom)
    res_ref[...] = res


_attn_call = pl.pallas_call(
    _attn_body,
    grid=(N // ROWBLK,),
    in_specs=[pl.BlockSpec((ROWBLK, HID), lambda i: (i, 0))] * 6 + [
        pl.BlockSpec((FILTER_NUM, K + 1), lambda i: (0, 0)),
        pl.BlockSpec((K + 1, K + 1), lambda i: (0, 0)),
        pl.BlockSpec((HID, HID), lambda i: (0, 0)),
        pl.BlockSpec((1, HID), lambda i: (0, 0)),
        pl.BlockSpec((HID, HID), lambda i: (0, 0)),
        pl.BlockSpec((1, HID), lambda i: (0, 0)),
    ],
    out_specs=pl.BlockSpec((ROWBLK, HID), lambda i: (i, 0)),
    out_shape=jax.ShapeDtypeStruct((N, HID), jnp.float32),
)


def _gram_body(a_ref, b_ref, o_ref):
    o_ref[...] = lax.dot_general(
        a_ref[...], b_ref[...], (((1,), (1,)), ((), ())),
        preferred_element_type=jnp.float32)


_gram_call = pl.pallas_call(
    _gram_body,
    grid=(N // GBLK,),
    in_specs=[
        pl.BlockSpec((GBLK, HID), lambda i: (i, 0)),
        pl.BlockSpec((N, HID), lambda i: (0, 0)),
    ],
    out_specs=pl.BlockSpec((GBLK, N), lambda i: (i, 0)),
    out_shape=jax.ShapeDtypeStruct((N, N), jnp.float32),
)


def kernel(x, edge_index, W1, b1, W2, b2, filt_w, Wf, bf, Wx, bx):
    pad = jnp.arange(EP - E, dtype=jnp.int32) % N
    rowp = jnp.concatenate([edge_index[0], pad]).reshape(EROWS, 128)
    colp = jnp.concatenate([edge_index[1], pad]).reshape(EROWS, 128)

    _sc_prep, _sc_round = _sc_kernels()
    scol, degp = _sc_prep(rowp, colp)
    h0, g, dinvb = _mlp_call(x, W1, b1.reshape(1, HID), W2,
                             b2.reshape(1, HID), degp)

    bs = [h0]
    for _ in range(K):
        p = _sc_round(g, rowp, scol)
        bn, g = _combine_call(bs[-1], p, dinvb)
        bs.append(bn)

    res = _attn_call(*bs, filt_w, jnp.asarray(_CM), Wf, bf.reshape(1, HID),
                     Wx, bx.reshape(1, HID))
    return _gram_call(res, res)
